# Initial kernel scaffold; baseline (speedup 1.0000x reference)
import jax
import jax.numpy as jnp
from jax.experimental import pallas as pl


def kernel(x, table):
    raise NotImplementedError("write your pallas kernel here")


# speedup vs baseline: 6.1105x; 6.1105x over previous
import functools

import jax
import jax.numpy as jnp
from jax import lax
from jax.experimental import pallas as pl
from jax.experimental.pallas import tpu as pltpu
from jax.experimental.pallas import tpu_sc as plsc

_OFFSET = 100
_LANES = 16
_IDXW = 128
_CHUNK = 1024


def kernel(x, table):
    B0, S = x.shape
    D = table.shape[1]
    B = B0 * S

    info = plsc.get_sparse_core_info()
    nw = info.num_cores * info.num_subcores
    b_per_w = B // nw
    assert B % nw == 0 and b_per_w % _CHUNK == 0
    n_chunks = b_per_w // _CHUNK
    rows_per_chunk = _CHUNK // _IDXW

    idx2d = x.reshape(B // _IDXW, _IDXW)
    mesh = plsc.VectorSubcoreMesh(core_axis_name="c", subcore_axis_name="s")

    @functools.partial(
        pl.kernel,
        mesh=mesh,
        out_type=jax.ShapeDtypeStruct((B, D), jnp.float32),
        scratch_types=[
            pltpu.VMEM((rows_per_chunk, _IDXW), jnp.int32),
            pltpu.VMEM((_CHUNK, D), jnp.float32),
            pltpu.SemaphoreType.DMA,
        ],
        compiler_params=pltpu.CompilerParams(use_tc_tiling_on_sc=False),
    )
    def emb(idx_hbm, table_hbm, out_hbm, idx_v, rows_v, sem):
        wid = lax.axis_index("s") * info.num_cores + lax.axis_index("c")
        row_base = wid * (b_per_w // _IDXW)
        out_base = wid * b_per_w

        def chunk_body(i, carry):
            pltpu.sync_copy(
                idx_hbm.at[pl.ds(row_base + i * rows_per_chunk, rows_per_chunk)],
                idx_v,
            )
            for r in range(rows_per_chunk):
                for l in range(_IDXW // _LANES):
                    sl = pl.ds(l * _LANES, _LANES)
                    idx_v[r, sl] = idx_v[r, sl] + _OFFSET
            copies = [
                pltpu.async_copy(
                    table_hbm.at[idx_v.at[r]],
                    rows_v.at[pl.ds(r * _IDXW, _IDXW)],
                    sem,
                )
                for r in range(rows_per_chunk)
            ]
            for c in copies:
                c.wait()
            pltpu.sync_copy(
                rows_v, out_hbm.at[pl.ds(out_base + i * _CHUNK, _CHUNK)]
            )
            return carry

        lax.fori_loop(0, n_chunks, chunk_body, 0)

    out = emb(idx2d, table)
    return out.reshape(B0, S, D)


# speedup vs baseline: 6.4580x; 1.0569x over previous
import functools

import jax
import jax.numpy as jnp
from jax import lax
from jax.experimental import pallas as pl
from jax.experimental.pallas import tpu as pltpu
from jax.experimental.pallas import tpu_sc as plsc

_OFFSET = 100
_LANES = 16
_IDXW = 128
_CHUNK = 1024
_NBUF = 2


def kernel(x, table):
    B0, S = x.shape
    D = table.shape[1]
    B = B0 * S

    info = plsc.get_sparse_core_info()
    nw = info.num_cores * info.num_subcores
    b_per_w = B // nw
    assert B % nw == 0 and b_per_w % (_CHUNK * _NBUF) == 0
    n_chunks = b_per_w // _CHUNK
    n_pairs = n_chunks // _NBUF
    nrc = _CHUNK // _IDXW

    idx2d = x.reshape(B // _IDXW, _IDXW)
    mesh = plsc.VectorSubcoreMesh(core_axis_name="c", subcore_axis_name="s")

    @functools.partial(
        pl.kernel,
        mesh=mesh,
        out_type=jax.ShapeDtypeStruct((B, D), jnp.float32),
        scratch_types=[
            pltpu.VMEM((_NBUF, nrc, _IDXW), jnp.int32),
            pltpu.VMEM((_NBUF, _CHUNK, D), jnp.float32),
            pltpu.SemaphoreType.DMA,
            pltpu.SemaphoreType.DMA,
            pltpu.SemaphoreType.DMA,
            pltpu.SemaphoreType.DMA,
            pltpu.SemaphoreType.DMA,
            pltpu.SemaphoreType.DMA,
        ],
        compiler_params=pltpu.CompilerParams(use_tc_tiling_on_sc=False),
    )
    def emb(idx_hbm, table_hbm, out_hbm, idx_v, rows_v,
            si0, si1, sg0, sg1, so0, so1):
        wid = lax.axis_index("s") * info.num_cores + lax.axis_index("c")
        row_base = wid * (b_per_w // _IDXW)
        out_base = wid * b_per_w
        si = (si0, si1)
        sg = (sg0, sg1)
        so = (so0, so1)

        def idx_copy(ci, b):
            return pltpu.make_async_copy(
                idx_hbm.at[pl.ds(row_base + ci * nrc, nrc)], idx_v.at[b], si[b])

        def out_copy(ci, b):
            return pltpu.make_async_copy(
                rows_v.at[b], out_hbm.at[pl.ds(out_base + ci * _CHUNK, _CHUNK)],
                so[b])

        idx_copy(0, 0).start()
        idx_copy(1, 1).start()

        def pair_body(p, carry):
            for b in range(_NBUF):
                ci = p * _NBUF + b

                @pl.when(p > 0)
                def _():
                    out_copy(ci - _NBUF, b).wait()

                idx_copy(ci, b).wait()

                gathers = []
                for r in range(nrc):
                    for l in range(_IDXW // _LANES):
                        sl = pl.ds(l * _LANES, _LANES)
                        idx_v[b, r, sl] = idx_v[b, r, sl] + _OFFSET
                    g = pltpu.make_async_copy(
                        table_hbm.at[idx_v.at[b].at[r]],
                        rows_v.at[b].at[pl.ds(r * _IDXW, _IDXW)],
                        sg[b])
                    g.start()
                    gathers.append(g)

                for g in gathers:
                    g.wait()
                out_copy(ci, b).start()

                @pl.when(p < n_pairs - 1)
                def _():
                    idx_copy(ci + _NBUF, b).start()
            return carry

        lax.fori_loop(0, n_pairs, pair_body, 0)
        out_copy(n_chunks - 2, 0).wait()
        out_copy(n_chunks - 1, 1).wait()

    out = emb(idx2d, table)
    return out.reshape(B0, S, D)
